# speedup vs baseline: 20.4865x; 20.4865x over previous
import jax
import jax.numpy as jnp
from jax.experimental import pallas as pl

_SAMPLES = 8
_K = 3


def _body(x_ref, s_ref, w_ref, b_ref, o_ref):
    xt = x_ref[0]
    S = s_ref[0]
    C, T = xt.shape
    M = S.shape[1]

    inner = jax.lax.dot_general(S, xt, (((0,), (0,)), ((), ())),
                                preferred_element_type=jnp.float32)
    s_sq = jnp.sum(S * S, axis=0)
    d = s_sq[:, None] - 2.0 * inner

    iota = jax.lax.broadcasted_iota(jnp.int32, (M, T), 0)
    ohs = []
    for _ in range(_K):
        mv = jnp.min(d, axis=0, keepdims=True)
        im = jnp.min(jnp.where(d == mv, iota, M), axis=0,
                     keepdims=True)
        oh = iota == im
        ohs.append(oh.astype(jnp.bfloat16))
        d = jnp.where(oh, jnp.inf, d)
    oh_all = jnp.concatenate(ohs, axis=0)

    ys = [jax.lax.dot_general(w_ref[k], S, (((1,), (0,)), ((), ())),
                              preferred_element_type=jnp.float32)
          for k in range(_K)]
    y_all = jnp.concatenate(ys, axis=1).astype(jnp.bfloat16)

    out = jax.lax.dot_general(y_all, oh_all, (((1,), (0,)), ((), ())),
                              preferred_element_type=jnp.float32)
    o_ref[0] = out + b_ref[...]


@jax.jit
def kernel(x, W, b):
    B, C, H, Wd = x.shape
    O = W.shape[0]
    N = H * Wd
    M = _SAMPLES * _SAMPLES

    x_ind = jnp.round(jnp.linspace(0, H - 1, _SAMPLES)).astype(jnp.int32)
    y_ind = jnp.round(jnp.linspace(0, Wd - 1, _SAMPLES)).astype(jnp.int32)
    xg, yg = jnp.meshgrid(x_ind, y_ind, indexing='ij')
    xs = x[:, :, xg, yg].reshape(B, C, M)
    x2 = x.reshape(B, C, N)
    Wt = jnp.transpose(W, (2, 0, 1))
    b2 = b.reshape(O, 1)

    T = next(t for t in (3584, 1792, 1024, 512, 256, 128) if N % t == 0)

    out = pl.pallas_call(
        _body,
        grid=(B, N // T),
        in_specs=[
            pl.BlockSpec((1, C, T), lambda bi, ti: (bi, 0, ti)),
            pl.BlockSpec((1, C, M), lambda bi, ti: (bi, 0, 0)),
            pl.BlockSpec((_K, O, C), lambda bi, ti: (0, 0, 0)),
            pl.BlockSpec((O, 1), lambda bi, ti: (0, 0)),
        ],
        out_specs=pl.BlockSpec((1, O, T), lambda bi, ti: (bi, 0, ti)),
        out_shape=jax.ShapeDtypeStruct((B, O, N), jnp.float32),
    )(x2, xs, Wt, b2)
    return out.reshape(B, O, H, Wd)


# speedup vs baseline: 32.5071x; 1.5868x over previous
import jax
import jax.numpy as jnp
import numpy as np
from jax.experimental import pallas as pl

_SAMPLES = 8
_K = 3


def _body(x_ref, s_ref, w_ref, b_ref, o_ref):
    xt = x_ref[0]
    S = s_ref[0]
    C, T = xt.shape
    M = S.shape[1]

    inner = jax.lax.dot_general(S, xt, (((0,), (0,)), ((), ())),
                                preferred_element_type=jnp.float32)
    s_sq = jnp.sum(S * S, axis=0)
    d = s_sq[:, None] - 2.0 * inner

    iota = jax.lax.broadcasted_iota(jnp.int32, (M, T), 0)
    ohs = []
    for _ in range(_K):
        mv = jnp.min(d, axis=0, keepdims=True)
        im = jnp.min(jnp.where(d == mv, iota, M), axis=0,
                     keepdims=True)
        oh = iota == im
        ohs.append(oh.astype(jnp.bfloat16))
        d = jnp.where(oh, jnp.inf, d)
    oh_all = jnp.concatenate(ohs, axis=0)

    ys = [jax.lax.dot_general(w_ref[k], S, (((1,), (0,)), ((), ())),
                              preferred_element_type=jnp.float32)
          for k in range(_K)]
    y_all = jnp.concatenate(ys, axis=1).astype(jnp.bfloat16)

    out = jax.lax.dot_general(y_all, oh_all, (((1,), (0,)), ((), ())),
                              preferred_element_type=jnp.float32)
    o_ref[0] = out + b_ref[...]


@jax.jit
def kernel(x, W, b):
    B, C, H, Wd = x.shape
    O = W.shape[0]
    N = H * Wd
    M = _SAMPLES * _SAMPLES

    x_ind = np.round(np.linspace(0, H - 1, _SAMPLES)).astype(np.int32)
    y_ind = np.round(np.linspace(0, Wd - 1, _SAMPLES)).astype(np.int32)
    cols = [x[:, :, int(h), int(w)] for h in x_ind for w in y_ind]
    xs = jnp.stack(cols, axis=-1)
    x2 = x.reshape(B, C, N)
    Wt = jnp.transpose(W, (2, 0, 1))
    b2 = b.reshape(O, 1)

    T = next(t for t in (3584, 1792, 1024, 512, 256, 128) if N % t == 0)

    out = pl.pallas_call(
        _body,
        grid=(B, N // T),
        in_specs=[
            pl.BlockSpec((1, C, T), lambda bi, ti: (bi, 0, ti)),
            pl.BlockSpec((1, C, M), lambda bi, ti: (bi, 0, 0)),
            pl.BlockSpec((_K, O, C), lambda bi, ti: (0, 0, 0)),
            pl.BlockSpec((O, 1), lambda bi, ti: (0, 0)),
        ],
        out_specs=pl.BlockSpec((1, O, T), lambda bi, ti: (bi, 0, ti)),
        out_shape=jax.ShapeDtypeStruct((B, O, N), jnp.float32),
    )(x2, xs, Wt, b2)
    return out.reshape(B, O, H, Wd)


# speedup vs baseline: 82.3698x; 2.5339x over previous
import jax
import jax.numpy as jnp
import numpy as np
from jax.experimental import pallas as pl

_SAMPLES = 8
_K = 3


def _make_sample_body(x_ind, y_ind):
    def body(*refs):
        row_refs = refs[:_SAMPLES]
        o_ref = refs[_SAMPLES]
        cols = []
        for r, h in zip(row_refs, x_ind):
            xt = r[0, :, int(h) % 8, :]
            cols.extend(xt[:, int(w):int(w) + 1] for w in y_ind)
        o_ref[0] = jnp.concatenate(cols, axis=1)
    return body


def _body(x_ref, s_ref, w_ref, b_ref, o_ref):
    C, Ht, Wd = x_ref.shape[1:]
    T = Ht * Wd
    xt = x_ref[0].reshape(C, T)
    S = s_ref[0]
    M = S.shape[1]

    inner = jax.lax.dot_general(S, xt, (((0,), (0,)), ((), ())),
                                preferred_element_type=jnp.float32)
    s_sq = jnp.sum(S * S, axis=0)
    d = s_sq[:, None] - 2.0 * inner

    iota = jax.lax.broadcasted_iota(jnp.int32, (M, T), 0)
    ohs = []
    for _ in range(_K):
        mv = jnp.min(d, axis=0, keepdims=True)
        im = jnp.min(jnp.where(d == mv, iota, M), axis=0,
                     keepdims=True)
        oh = iota == im
        ohs.append(oh.astype(jnp.bfloat16))
        d = jnp.where(oh, jnp.inf, d)
    oh_all = jnp.concatenate(ohs, axis=0)

    ys = [jax.lax.dot_general(w_ref[k], S, (((1,), (0,)), ((), ())),
                              preferred_element_type=jnp.float32)
          for k in range(_K)]
    y_all = jnp.concatenate(ys, axis=1).astype(jnp.bfloat16)

    out = jax.lax.dot_general(y_all, oh_all, (((1,), (0,)), ((), ())),
                              preferred_element_type=jnp.float32)
    out = out + b_ref[...]
    o_ref[0] = out.reshape(out.shape[0], Ht, Wd)


@jax.jit
def kernel(x, W, b):
    B, C, H, Wd = x.shape
    O = W.shape[0]
    M = _SAMPLES * _SAMPLES

    x_ind = np.round(np.linspace(0, H - 1, _SAMPLES)).astype(np.int32)
    y_ind = np.round(np.linspace(0, Wd - 1, _SAMPLES)).astype(np.int32)
    row_specs = [
        pl.BlockSpec((1, C, 8, Wd), lambda bi, h=int(h): (bi, 0, h // 8, 0))
        for h in x_ind
    ]
    xs = pl.pallas_call(
        _make_sample_body(x_ind, y_ind),
        grid=(B,),
        in_specs=row_specs,
        out_specs=pl.BlockSpec((1, C, M), lambda bi: (bi, 0, 0)),
        out_shape=jax.ShapeDtypeStruct((B, C, M), jnp.float32),
    )(*([x] * _SAMPLES))

    Wt = jnp.transpose(W, (2, 0, 1))
    b2 = b.reshape(O, 1)

    Ht = next(t for t in (16, 8, 4, 2, 1) if H % t == 0)

    return pl.pallas_call(
        _body,
        grid=(B, H // Ht),
        in_specs=[
            pl.BlockSpec((1, C, Ht, Wd), lambda bi, ti: (bi, 0, ti, 0)),
            pl.BlockSpec((1, C, M), lambda bi, ti: (bi, 0, 0)),
            pl.BlockSpec((_K, O, C), lambda bi, ti: (0, 0, 0)),
            pl.BlockSpec((O, 1), lambda bi, ti: (0, 0)),
        ],
        out_specs=pl.BlockSpec((1, O, Ht, Wd), lambda bi, ti: (bi, 0, ti, 0)),
        out_shape=jax.ShapeDtypeStruct((B, O, H, Wd), jnp.float32),
    )(x, xs, Wt, b2)


# speedup vs baseline: 89.9726x; 1.0923x over previous
import jax
import jax.numpy as jnp
import numpy as np
from jax.experimental import pallas as pl

_SAMPLES = 8
_K = 3


def _make_sample_body(x_ind, y_ind):
    def body(*refs):
        row_refs = refs[:_SAMPLES]
        o_ref = refs[_SAMPLES]
        cols = []
        for r, h in zip(row_refs, x_ind):
            xt = r[0, :, int(h) % 8, :]
            cols.extend(xt[:, int(w):int(w) + 1] for w in y_ind)
        o_ref[0] = jnp.concatenate(cols, axis=1)
    return body


def _body(x_ref, s_ref, w_ref, b_ref, o_ref):
    C, Ht, Wd = x_ref.shape[1:]
    T = Ht * Wd
    xt = x_ref[0].reshape(C, T)
    S = s_ref[0]
    M = S.shape[1]

    inner = jax.lax.dot_general(S, xt, (((0,), (0,)), ((), ())),
                                preferred_element_type=jnp.float32)
    s_sq = jnp.sum(S * S, axis=0)
    d = s_sq[:, None] - 2.0 * inner

    iota = jax.lax.broadcasted_iota(jnp.int32, (M, T), 0).astype(jnp.float32)
    ohs = []
    for _ in range(_K):
        mv = jnp.min(d, axis=0, keepdims=True)
        f = jnp.where(d == mv, iota, jnp.float32(M))
        imf = jnp.min(f, axis=0, keepdims=True)
        oh = f == imf
        ohs.append(oh.astype(jnp.bfloat16))
        d = jnp.where(oh, jnp.inf, d)
    oh_all = jnp.concatenate(ohs, axis=0)

    ys = [jax.lax.dot_general(w_ref[k], S, (((1,), (0,)), ((), ())),
                              preferred_element_type=jnp.float32)
          for k in range(_K)]
    y_all = jnp.concatenate(ys, axis=1).astype(jnp.bfloat16)

    out = jax.lax.dot_general(y_all, oh_all, (((1,), (0,)), ((), ())),
                              preferred_element_type=jnp.float32)
    out = out + b_ref[...]
    o_ref[0] = out.reshape(out.shape[0], Ht, Wd)


@jax.jit
def kernel(x, W, b):
    B, C, H, Wd = x.shape
    O = W.shape[0]
    M = _SAMPLES * _SAMPLES

    x_ind = np.round(np.linspace(0, H - 1, _SAMPLES)).astype(np.int32)
    y_ind = np.round(np.linspace(0, Wd - 1, _SAMPLES)).astype(np.int32)
    row_specs = [
        pl.BlockSpec((1, C, 8, Wd), lambda bi, h=int(h): (bi, 0, h // 8, 0))
        for h in x_ind
    ]
    xs = pl.pallas_call(
        _make_sample_body(x_ind, y_ind),
        grid=(B,),
        in_specs=row_specs,
        out_specs=pl.BlockSpec((1, C, M), lambda bi: (bi, 0, 0)),
        out_shape=jax.ShapeDtypeStruct((B, C, M), jnp.float32),
    )(*([x] * _SAMPLES))

    Wt = jnp.transpose(W, (2, 0, 1))
    b2 = b.reshape(O, 1)

    Ht = next(t for t in (32, 16, 8, 4, 2, 1) if H % t == 0)

    return pl.pallas_call(
        _body,
        grid=(B, H // Ht),
        in_specs=[
            pl.BlockSpec((1, C, Ht, Wd), lambda bi, ti: (bi, 0, ti, 0)),
            pl.BlockSpec((1, C, M), lambda bi, ti: (bi, 0, 0)),
            pl.BlockSpec((_K, O, C), lambda bi, ti: (0, 0, 0)),
            pl.BlockSpec((O, 1), lambda bi, ti: (0, 0)),
        ],
        out_specs=pl.BlockSpec((1, O, Ht, Wd), lambda bi, ti: (bi, 0, ti, 0)),
        out_shape=jax.ShapeDtypeStruct((B, O, H, Wd), jnp.float32),
    )(x, xs, Wt, b2)


# speedup vs baseline: 92.6652x; 1.0299x over previous
import jax
import jax.numpy as jnp
import numpy as np
from jax.experimental import pallas as pl
from jax.experimental.pallas import tpu as pltpu

_SAMPLES = 8
_K = 3


def _make_body(B, C, O, H, Wd, HT, x_rows, y_cols):
    TPB = H // HT
    M = _SAMPLES * _SAMPLES

    def body(x_ref, x_hbm, w_ref, b_ref, o_ref,
             sbuf, s_ref, ssq_ref, y_ref, smp_sem):
        bi = pl.program_id(0)
        ti = pl.program_id(1)

        def sample_copies():
            return [pltpu.make_async_copy(
                x_hbm.at[bb, :, h, :], sbuf.at[bb, r], smp_sem)
                for bb in range(B) for r, h in enumerate(x_rows)]

        @pl.when((bi == 0) & (ti == 0))
        def _fetch_samples():
            for cp in sample_copies():
                cp.start()
            for cp in sample_copies():
                cp.wait()

        @pl.when(ti == 0)
        def _per_batch():
            cols = []
            for r in range(_SAMPLES):
                row = sbuf[bi, r]
                cols.extend(row[:, w:w + 1] for w in y_cols)
            Sv = jnp.concatenate(cols, axis=1)
            s_ref[...] = Sv
            ssq_ref[...] = jnp.sum(Sv * Sv, axis=0, keepdims=True).T
            ys = [jax.lax.dot_general(w_ref[k], Sv, (((1,), (0,)), ((), ())),
                                      preferred_element_type=jnp.float32)
                  for k in range(_K)]
            y_ref[...] = jnp.concatenate(ys, axis=1).astype(jnp.bfloat16)

        T = HT * Wd
        xt = x_ref[0].reshape(C, T)
        Sv = s_ref[...]

        inner = jax.lax.dot_general(Sv, xt, (((0,), (0,)), ((), ())),
                                    preferred_element_type=jnp.float32)
        d = ssq_ref[...] - 2.0 * inner

        iota = jax.lax.broadcasted_iota(
            jnp.int32, (M, T), 0).astype(jnp.float32)
        ohs = []
        for k in range(_K):
            mv = jnp.min(d, axis=0, keepdims=True)
            f = jnp.where(d == mv, iota, jnp.float32(M))
            imf = jnp.min(f, axis=0, keepdims=True)
            oh = f == imf
            ohs.append(oh.astype(jnp.bfloat16))
            if k < _K - 1:
                d = jnp.where(oh, jnp.inf, d)
        oh_all = jnp.concatenate(ohs, axis=0)

        out = jax.lax.dot_general(y_ref[...], oh_all,
                                  (((1,), (0,)), ((), ())),
                                  preferred_element_type=jnp.float32)
        out = out + b_ref[...]
        o_ref[0] = out.reshape(O, HT, Wd)

    return body


@jax.jit
def kernel(x, W, b):
    B, C, H, Wd = x.shape
    O = W.shape[0]
    M = _SAMPLES * _SAMPLES

    x_rows = [int(h) for h in
              np.round(np.linspace(0, H - 1, _SAMPLES)).astype(np.int32)]
    y_cols = [int(w) for w in
              np.round(np.linspace(0, Wd - 1, _SAMPLES)).astype(np.int32)]

    HT = next(t for t in (32, 16, 8, 4, 2, 1) if H % t == 0)
    body = _make_body(B, C, O, H, Wd, HT, x_rows, y_cols)

    Wt = jnp.transpose(W, (2, 0, 1))
    b2 = b.reshape(O, 1)

    return pl.pallas_call(
        body,
        grid=(B, H // HT),
        in_specs=[
            pl.BlockSpec((1, C, HT, Wd), lambda bi, ti: (bi, 0, ti, 0)),
            pl.BlockSpec(memory_space=pl.ANY),
            pl.BlockSpec((_K, O, C), lambda bi, ti: (0, 0, 0)),
            pl.BlockSpec((O, 1), lambda bi, ti: (0, 0)),
        ],
        out_specs=pl.BlockSpec((1, O, HT, Wd), lambda bi, ti: (bi, 0, ti, 0)),
        out_shape=jax.ShapeDtypeStruct((B, O, H, Wd), jnp.float32),
        scratch_shapes=[
            pltpu.VMEM((B, _SAMPLES, C, Wd), jnp.float32),
            pltpu.VMEM((C, M), jnp.float32),
            pltpu.VMEM((M, 1), jnp.float32),
            pltpu.VMEM((O, _K * M), jnp.bfloat16),
            pltpu.SemaphoreType.DMA,
        ],
    )(x, x, Wt, b2)


# speedup vs baseline: 94.4922x; 1.0197x over previous
import jax
import jax.numpy as jnp
import numpy as np
from jax.experimental import pallas as pl
from jax.experimental.pallas import tpu as pltpu

_SAMPLES = 8
_K = 3


def _make_body(B, C, O, H, Wd, HT, x_rows, y_cols):
    TPB = H // HT
    M = _SAMPLES * _SAMPLES

    def body(x_ref, x_hbm, w_ref, b_ref, o_ref,
             sbuf, s_ref, ssq_ref, y_ref, smp_sem):
        bi = pl.program_id(0)
        ti = pl.program_id(1)

        def sample_copies():
            return [pltpu.make_async_copy(
                x_hbm.at[bb, :, h, :], sbuf.at[bb, r], smp_sem)
                for bb in range(B) for r, h in enumerate(x_rows)]

        @pl.when((bi == 0) & (ti == 0))
        def _fetch_samples():
            for cp in sample_copies():
                cp.start()
            for cp in sample_copies():
                cp.wait()

        @pl.when(ti == 0)
        def _per_batch():
            cols = []
            for r in range(_SAMPLES):
                row = sbuf[bi, r]
                cols.extend(row[:, w:w + 1] for w in y_cols)
            Sv = jnp.concatenate(cols, axis=1)
            s_ref[...] = Sv
            ssq_ref[...] = jnp.sum(Sv * Sv, axis=0, keepdims=True).T
            ys = [jax.lax.dot_general(w_ref[k], Sv, (((1,), (0,)), ((), ())),
                                      preferred_element_type=jnp.float32)
                  for k in range(_K)]
            y_ref[...] = jnp.concatenate(ys, axis=1).astype(jnp.bfloat16)

        T = HT * Wd
        xt = x_ref[0].reshape(C, T)
        Sv = s_ref[...]

        inner = jax.lax.dot_general(Sv, xt, (((0,), (0,)), ((), ())),
                                    preferred_element_type=jnp.float32)
        d = ssq_ref[...] - 2.0 * inner

        iota = jax.lax.broadcasted_iota(
            jnp.int32, (M, T), 0).astype(jnp.float32)
        ohs = []
        for k in range(_K):
            mv = jnp.min(d, axis=0, keepdims=True)
            f = jnp.where(d == mv, iota, jnp.float32(M))
            imf = jnp.min(f, axis=0, keepdims=True)
            oh = f == imf
            ohs.append(oh.astype(jnp.bfloat16))
            if k < _K - 1:
                d = jnp.where(oh, jnp.inf, d)
        oh_all = jnp.concatenate(ohs, axis=0)

        out = jax.lax.dot_general(y_ref[...], oh_all,
                                  (((1,), (0,)), ((), ())),
                                  preferred_element_type=jnp.float32)
        out = out + b_ref[...]
        o_ref[0] = out.reshape(O, HT, Wd)

    return body


@jax.jit
def kernel(x, W, b):
    B, C, H, Wd = x.shape
    O = W.shape[0]
    M = _SAMPLES * _SAMPLES

    x_rows = [int(h) for h in
              np.round(np.linspace(0, H - 1, _SAMPLES)).astype(np.int32)]
    y_cols = [int(w) for w in
              np.round(np.linspace(0, Wd - 1, _SAMPLES)).astype(np.int32)]

    HT = next(t for t in (56, 32, 16, 8, 4, 2, 1) if H % t == 0)
    body = _make_body(B, C, O, H, Wd, HT, x_rows, y_cols)

    Wt = jnp.transpose(W, (2, 0, 1))
    b2 = b.reshape(O, 1)

    return pl.pallas_call(
        body,
        grid=(B, H // HT),
        in_specs=[
            pl.BlockSpec((1, C, HT, Wd), lambda bi, ti: (bi, 0, ti, 0)),
            pl.BlockSpec(memory_space=pl.ANY),
            pl.BlockSpec((_K, O, C), lambda bi, ti: (0, 0, 0)),
            pl.BlockSpec((O, 1), lambda bi, ti: (0, 0)),
        ],
        out_specs=pl.BlockSpec((1, O, HT, Wd), lambda bi, ti: (bi, 0, ti, 0)),
        out_shape=jax.ShapeDtypeStruct((B, O, H, Wd), jnp.float32),
        scratch_shapes=[
            pltpu.VMEM((B, _SAMPLES, C, Wd), jnp.float32),
            pltpu.VMEM((C, M), jnp.float32),
            pltpu.VMEM((M, 1), jnp.float32),
            pltpu.VMEM((O, _K * M), jnp.bfloat16),
            pltpu.SemaphoreType.DMA,
        ],
    )(x, x, Wt, b2)
